# baseline (speedup 1.0000x reference)
import jax
import jax.numpy as jnp
from jax import lax
from jax.experimental import pallas as pl
from jax.experimental.pallas import tpu as pltpu
from jax.experimental.pallas import tpu_sc as plsc

D_MODEL = 1024
D_HIDDEN = 4096
N_EXP = 8
T_BLK = 512
BT = 512
T_TOT = 4096
NSLOT = T_TOT * 2
P_PAD = NSLOT + N_EXP * BT
NB = P_PAD // BT
NSCHED = 32

NSC = 2
NTILE = 16
LANES = 16


def _router_body(x_ref, gw_ref, eidx_ref, wts_ref, aux_ref, p_acc, f_acc):
    j = pl.program_id(0)
    nblk = pl.num_programs(0)
    logits = jnp.dot(x_ref[...], gw_ref[...], preferred_element_type=jnp.float32)
    m = jnp.max(logits, axis=-1, keepdims=True)
    ex = jnp.exp(logits - m)
    probs = ex / jnp.sum(ex, axis=-1, keepdims=True)
    idx = lax.broadcasted_iota(jnp.int32, probs.shape, 1)
    m1 = jnp.max(probs, axis=-1, keepdims=True)
    a1 = jnp.min(jnp.where(probs == m1, idx, N_EXP), axis=-1, keepdims=True)
    mask1 = (idx == a1).astype(jnp.float32)
    pm = jnp.where(mask1 > 0, -1.0, probs)
    m2 = jnp.max(pm, axis=-1, keepdims=True)
    a2 = jnp.min(jnp.where(pm == m2, idx, N_EXP), axis=-1, keepdims=True)
    mask2 = (idx == a2).astype(jnp.float32)
    den = m1 + m2
    eidx_ref[...] = jnp.concatenate([a1, a2], axis=1)
    wts_ref[...] = jnp.concatenate([m1 / den, m2 / den], axis=1)

    @pl.when(j == 0)
    def _():
        p_acc[...] = jnp.zeros_like(p_acc)
        f_acc[...] = jnp.zeros_like(f_acc)

    p_acc[...] += probs
    f_acc[...] += mask1 + mask2

    @pl.when(j == nblk - 1)
    def _():
        t_tot = nblk * T_BLK
        p_sum = jnp.sum(p_acc[...], axis=0)
        f_sum = jnp.sum(f_acc[...], axis=0)
        aux_ref[0, 0] = jnp.sum(p_sum * f_sum) * (N_EXP / (t_tot * t_tot))


def _run_router(x_flat, gate_w):
    return pl.pallas_call(
        _router_body,
        grid=(T_TOT // T_BLK,),
        in_specs=[
            pl.BlockSpec((T_BLK, D_MODEL), lambda j: (j, 0)),
            pl.BlockSpec((D_MODEL, N_EXP), lambda j: (0, 0)),
        ],
        out_specs=[
            pl.BlockSpec((T_BLK, 2), lambda j: (j, 0)),
            pl.BlockSpec((T_BLK, 2), lambda j: (j, 0)),
            pl.BlockSpec(memory_space=pltpu.SMEM, block_shape=(1, 1),
                         index_map=lambda j: (0, 0)),
        ],
        out_shape=[
            jax.ShapeDtypeStruct((T_TOT, 2), jnp.int32),
            jax.ShapeDtypeStruct((T_TOT, 2), jnp.float32),
            jax.ShapeDtypeStruct((1, 1), jnp.float32),
        ],
        scratch_shapes=[
            pltpu.VMEM((T_BLK, N_EXP), jnp.float32),
            pltpu.VMEM((T_BLK, N_EXP), jnp.float32),
        ],
        compiler_params=pltpu.CompilerParams(
            dimension_semantics=("arbitrary",)),
    )(x_flat, gate_w)


_NW = NSC * NTILE
_CHUNK = NSLOT // _NW
_NROW = _CHUNK // 128


def _sc_take(a, idx):
    return lax.gather(
        a, idx[:, None],
        dimension_numbers=lax.GatherDimensionNumbers(
            offset_dims=(), collapsed_slice_dims=(0,), start_index_map=(0,)),
        slice_sizes=(1,),
        mode=lax.GatherScatterMode.PROMISE_IN_BOUNDS)


def _sc_splat(v, i):
    return _sc_take(v, jnp.full((16,), i, jnp.int32))


def _sc_treesum(v, lane):
    for d in (1, 2, 4, 8):
        v = v + _sc_take(v, lane ^ d)
    return v


def _sc_cumsum(v, lane):
    for d in (1, 2, 4, 8):
        g = _sc_take(v, jnp.maximum(lane - d, 0))
        v = v + jnp.where(lane >= d, g, 0)
    return v


def _sort_body(ef_hbm, wf_hbm, st_hbm, sw_hbm, posf_hbm, sex_hbm, ssk_hbm,
               ev, wv2, posb2, tokb2, cnt2, schedb, schedsk, sem1, sem2):
    wid = lax.axis_index("s") * NSC + lax.axis_index("c")
    lane = lax.iota(jnp.int32, 16)
    zeros_i = jnp.zeros((16,), jnp.int32)

    pltpu.sync_copy(ef_hbm, ev)
    sbase = wid * _CHUNK
    for r in range(_NROW):
        pltpu.sync_copy(wf_hbm.at[pl.ds(sbase + r * 128, 128)], wv2.at[r])

    for e in range(2 * N_EXP):
        cnt2[e] = zeros_i
    pgroups = zeros_i + wid * (_CHUNK // 16)

    def hbody(g, c):
        v = ev[pl.ds(g * 16, 16)]
        gv = zeros_i + g
        mpre = jnp.minimum(jnp.maximum(pgroups - gv, 0), 1)
        for e in range(N_EXP):
            inc = jnp.where(v == e, 1, 0)
            cnt2[e] = cnt2[e] + inc
            cnt2[N_EXP + e] = cnt2[N_EXP + e] + inc * mpre
        return c
    lax.fori_loop(0, NSLOT // 16, hbody, 0)

    c_tot = zeros_i
    pre = zeros_i
    for e in range(N_EXP):
        c_tot = c_tot + jnp.where(lane == e, _sc_treesum(cnt2[e], lane), 0)
        pre = pre + jnp.where(lane == e, _sc_treesum(cnt2[N_EXP + e], lane), 0)
    pad = (c_tot + (BT - 1)) & jnp.int32(-BT)
    incl = _sc_cumsum(pad, lane)
    poff = incl - pad
    base = poff + pre

    running = base
    for g in range(_CHUNK // 16):
        v = ev[pl.ds(sbase + g * 16, 16)]
        posv = zeros_i
        for e in range(N_EXP):
            mk = (v == e)
            ones = jnp.where(mk, 1, 0)
            cs = _sc_cumsum(ones, lane)
            be = _sc_splat(running, e)
            tot = _sc_splat(cs, 15)
            posv = jnp.where(mk, cs - 1 + be, posv)
            running = running + jnp.where(lane == e, tot, 0)
        posb2[g // 8, pl.ds((g % 8) * 16, 16)] = posv
        tokb2[g // 8, pl.ds((g % 8) * 16, 16)] = (lane + (sbase + g * 16)) >> 1

    for r in range(_NROW):
        pltpu.sync_copy(posb2.at[r], posf_hbm.at[pl.ds(sbase + r * 128, 128)])
        pltpu.async_copy(tokb2.at[r], st_hbm.at[posb2.at[r]], sem1).wait()
        pltpu.async_copy(wv2.at[r], sw_hbm.at[posb2.at[r]], sem2).wait()

    @pl.when(wid == 0)
    def _():
        end7 = _sc_splat(incl, N_EXP - 1)
        for vb in range(NSCHED // 16):
            bidx = lane + vb * 16
            v0 = bidx * BT
            exv = jnp.zeros((16,), jnp.int32)
            for e in range(N_EXP):
                end_e = _sc_splat(incl, e)
                exv = exv + jnp.where(v0 >= end_e, 1, 0)
            schedb[pl.ds(vb * 16, 16)] = jnp.minimum(exv, N_EXP - 1)
            schedsk[pl.ds(vb * 16, 16)] = jnp.where(v0 >= end7, 1, 0)
        pltpu.sync_copy(schedb, sex_hbm)
        pltpu.sync_copy(schedsk, ssk_hbm)


def _run_sort(ef, wf):
    mesh = plsc.VectorSubcoreMesh(core_axis_name="c", subcore_axis_name="s")
    return pl.kernel(
        _sort_body,
        out_type=[
            jax.ShapeDtypeStruct((P_PAD,), jnp.int32),
            jax.ShapeDtypeStruct((P_PAD,), jnp.float32),
            jax.ShapeDtypeStruct((NSLOT,), jnp.int32),
            jax.ShapeDtypeStruct((NSCHED,), jnp.int32),
            jax.ShapeDtypeStruct((NSCHED,), jnp.int32),
        ],
        mesh=mesh,
        scratch_types=[
            pltpu.VMEM((NSLOT,), jnp.int32),
            pltpu.VMEM((_NROW, 128), jnp.float32),
            pltpu.VMEM((_NROW, 128), jnp.int32),
            pltpu.VMEM((_NROW, 128), jnp.int32),
            pltpu.VMEM((2 * N_EXP, 16), jnp.int32),
            pltpu.VMEM((NSCHED,), jnp.int32),
            pltpu.VMEM((NSCHED,), jnp.int32),
            pltpu.SemaphoreType.DMA,
            pltpu.SemaphoreType.DMA,
        ],
    )(ef, wf)


_GROWS = P_PAD // (NSC * NTILE)


def _gather_body(st_hbm, x_hbm, xs_hbm, idx0, idx1, rows0, rows1,
                 sem0, sem1, wsem0, wsem1):
    wid = lax.axis_index("s") * NSC + lax.axis_index("c")
    base = wid * _GROWS
    nch = _GROWS // 32
    bufs = [(idx0, rows0, sem0, wsem0), (idx1, rows1, sem1, wsem1)]

    def fire(j):
        idxv, rowsv, sem, _ = bufs[j % 2]
        pltpu.sync_copy(st_hbm.at[pl.ds(base + j * 32, 32)], idxv)
        idxv[...] = jnp.minimum(jnp.maximum(idxv[...], 0), T_TOT - 1)
        return pltpu.async_copy(x_hbm.at[idxv], rowsv, sem)

    hs = {0: fire(0)}
    ws = {}
    for j in range(nch):
        if j + 1 < nch:
            if j - 1 >= 0:
                ws[j - 1].wait()
            hs[j + 1] = fire(j + 1)
        hs[j].wait()
        rowsv, wsem = bufs[j % 2][1], bufs[j % 2][3]
        ws[j] = pltpu.async_copy(rowsv, xs_hbm.at[pl.ds(base + j * 32, 32)],
                                 wsem)
    ws[nch - 2].wait()
    ws[nch - 1].wait()


def _run_gather(st, x_flat):
    mesh = plsc.VectorSubcoreMesh(core_axis_name="c", subcore_axis_name="s")
    return pl.kernel(
        _gather_body,
        out_type=jax.ShapeDtypeStruct((P_PAD, D_MODEL), jnp.float32),
        mesh=mesh,
        scratch_types=[
            pltpu.VMEM((32,), jnp.int32),
            pltpu.VMEM((32,), jnp.int32),
            pltpu.VMEM((32, D_MODEL), jnp.float32),
            pltpu.VMEM((32, D_MODEL), jnp.float32),
            pltpu.SemaphoreType.DMA,
            pltpu.SemaphoreType.DMA,
            pltpu.SemaphoreType.DMA,
            pltpu.SemaphoreType.DMA,
        ],
    )(st, x_flat)


def _ffn_body(sex_ref, ssk_ref, xs_ref, sw_ref, w1_ref, b1_ref, w2_ref,
              b2_ref, out_ref):
    b = pl.program_id(0)

    @pl.when(ssk_ref[b] == 0)
    def _():
        xb = xs_ref[...].astype(jnp.bfloat16)
        h = jnp.dot(xb, w1_ref[0], preferred_element_type=jnp.float32)
        h = h + b1_ref[0]
        h = h * jax.nn.sigmoid(h)
        o = jnp.dot(h.astype(jnp.bfloat16), w2_ref[0],
                    preferred_element_type=jnp.float32)
        out_ref[...] = (o + b2_ref[0]) * sw_ref[...]

    @pl.when(ssk_ref[b] != 0)
    def _():
        out_ref[...] = jnp.zeros_like(out_ref)


def _run_ffn(sex, ssk, xs, sw2d, w1_bf, b1r, w2_bf, b2r):
    grid_spec = pltpu.PrefetchScalarGridSpec(
        num_scalar_prefetch=2,
        grid=(NB,),
        in_specs=[
            pl.BlockSpec((BT, D_MODEL), lambda b, sex, ssk: (b, 0)),
            pl.BlockSpec((BT, 1), lambda b, sex, ssk: (b, 0)),
            pl.BlockSpec((1, D_MODEL, D_HIDDEN),
                         lambda b, sex, ssk: (sex[b], 0, 0)),
            pl.BlockSpec((1, 1, D_HIDDEN), lambda b, sex, ssk: (sex[b], 0, 0)),
            pl.BlockSpec((1, D_HIDDEN, D_MODEL),
                         lambda b, sex, ssk: (sex[b], 0, 0)),
            pl.BlockSpec((1, 1, D_MODEL), lambda b, sex, ssk: (sex[b], 0, 0)),
        ],
        out_specs=pl.BlockSpec((BT, D_MODEL), lambda b, sex, ssk: (b, 0)),
    )
    return pl.pallas_call(
        _ffn_body,
        grid_spec=grid_spec,
        out_shape=jax.ShapeDtypeStruct((P_PAD, D_MODEL), jnp.float32),
        compiler_params=pltpu.CompilerParams(
            dimension_semantics=("arbitrary",)),
    )(sex, ssk, xs, sw2d, w1_bf, b1r, w2_bf, b2r)


_CTOK = T_TOT // (NSC * NTILE)


def _combine_body(posf_hbm, os_hbm, out_hbm, pv0, pv1, rows0, rows1, outv,
                  sem0, sem1):
    wid = lax.axis_index("s") * NSC + lax.axis_index("c")
    tbase = wid * _CTOK
    pbase = wid * _CTOK * 2
    nch = _CTOK // 16
    bufs = [(pv0, rows0, sem0), (pv1, rows1, sem1)]

    def fire(j, buf):
        pv, rowsv, sem = buf
        pltpu.sync_copy(posf_hbm.at[pl.ds(pbase + j * 32, 32)], pv)
        return pltpu.async_copy(os_hbm.at[pv], rowsv, sem)

    hs = {0: fire(0, bufs[0])}
    for j in range(nch):
        if j + 1 < nch:
            hs[j + 1] = fire(j + 1, bufs[(j + 1) % 2])
        hs[j].wait()
        rowsv = bufs[j % 2][1]

        def row_body(r, carry):
            r2 = 2 * r
            for c in range(D_MODEL // 16):
                a = rowsv[r2, pl.ds(c * 16, 16)]
                bb = rowsv[r2 + 1, pl.ds(c * 16, 16)]
                outv[r, pl.ds(c * 16, 16)] = a + bb
            return carry

        lax.fori_loop(0, 16, row_body, 0)
        pltpu.sync_copy(outv, out_hbm.at[pl.ds(tbase + j * 16, 16)])


def _run_combine(posf, os):
    mesh = plsc.VectorSubcoreMesh(core_axis_name="c", subcore_axis_name="s")
    return pl.kernel(
        _combine_body,
        out_type=jax.ShapeDtypeStruct((T_TOT, D_MODEL), jnp.float32),
        mesh=mesh,
        scratch_types=[
            pltpu.VMEM((32,), jnp.int32),
            pltpu.VMEM((32,), jnp.int32),
            pltpu.VMEM((32, D_MODEL), jnp.float32),
            pltpu.VMEM((32, D_MODEL), jnp.float32),
            pltpu.VMEM((16, D_MODEL), jnp.float32),
            pltpu.SemaphoreType.DMA,
            pltpu.SemaphoreType.DMA,
        ],
    )(posf, os)


def kernel(x, gate_w, w1, b1, w2, b2):
    B, S, D = x.shape
    x_flat = x.reshape(B * S, D)

    eidx, wts, aux = _run_router(x_flat, gate_w)
    ef = eidx.reshape(NSLOT)
    wf = wts.reshape(NSLOT)

    st, sw, posf, sex, ssk = _run_sort(ef, wf)

    xs = _run_gather(st, x_flat)

    os_ = _run_ffn(sex, ssk, xs, sw.reshape(P_PAD, 1),
                   w1.astype(jnp.bfloat16), b1.reshape(N_EXP, 1, D_HIDDEN),
                   w2.astype(jnp.bfloat16), b2.reshape(N_EXP, 1, D_MODEL))

    out_flat = _run_combine(posf, os_)
    return out_flat.reshape(B, S, D), aux[0, 0]


# speedup vs baseline: 1.2185x; 1.2185x over previous
import jax
import jax.numpy as jnp
from jax import lax
from jax.experimental import pallas as pl
from jax.experimental.pallas import tpu as pltpu
from jax.experimental.pallas import tpu_sc as plsc

D_MODEL = 1024
D_HIDDEN = 4096
N_EXP = 8
T_BLK = 512
BT = 128
T_TOT = 4096
NSLOT = T_TOT * 2
P_PAD = NSLOT + N_EXP * BT
NB = P_PAD // BT
NSCHED = 80

NSC = 2
NTILE = 16
LANES = 16


def _router_body(x_ref, gw_ref, eidx_ref, wts_ref, aux_ref, p_acc, f_acc):
    j = pl.program_id(0)
    nblk = pl.num_programs(0)
    logits = jnp.dot(x_ref[...], gw_ref[...], preferred_element_type=jnp.float32)
    m = jnp.max(logits, axis=-1, keepdims=True)
    ex = jnp.exp(logits - m)
    probs = ex / jnp.sum(ex, axis=-1, keepdims=True)
    idx = lax.broadcasted_iota(jnp.int32, probs.shape, 1)
    m1 = jnp.max(probs, axis=-1, keepdims=True)
    a1 = jnp.min(jnp.where(probs == m1, idx, N_EXP), axis=-1, keepdims=True)
    mask1 = (idx == a1).astype(jnp.float32)
    pm = jnp.where(mask1 > 0, -1.0, probs)
    m2 = jnp.max(pm, axis=-1, keepdims=True)
    a2 = jnp.min(jnp.where(pm == m2, idx, N_EXP), axis=-1, keepdims=True)
    mask2 = (idx == a2).astype(jnp.float32)
    den = m1 + m2
    eidx_ref[...] = jnp.concatenate([a1, a2], axis=1)
    wts_ref[...] = jnp.concatenate([m1 / den, m2 / den], axis=1)

    @pl.when(j == 0)
    def _():
        p_acc[...] = jnp.zeros_like(p_acc)
        f_acc[...] = jnp.zeros_like(f_acc)

    p_acc[...] += probs
    f_acc[...] += mask1 + mask2

    @pl.when(j == nblk - 1)
    def _():
        t_tot = nblk * T_BLK
        p_sum = jnp.sum(p_acc[...], axis=0)
        f_sum = jnp.sum(f_acc[...], axis=0)
        aux_ref[0, 0] = jnp.sum(p_sum * f_sum) * (N_EXP / (t_tot * t_tot))


def _run_router(x_flat, gate_w):
    return pl.pallas_call(
        _router_body,
        grid=(T_TOT // T_BLK,),
        in_specs=[
            pl.BlockSpec((T_BLK, D_MODEL), lambda j: (j, 0)),
            pl.BlockSpec((D_MODEL, N_EXP), lambda j: (0, 0)),
        ],
        out_specs=[
            pl.BlockSpec((T_BLK, 2), lambda j: (j, 0)),
            pl.BlockSpec((T_BLK, 2), lambda j: (j, 0)),
            pl.BlockSpec(memory_space=pltpu.SMEM, block_shape=(1, 1),
                         index_map=lambda j: (0, 0)),
        ],
        out_shape=[
            jax.ShapeDtypeStruct((T_TOT, 2), jnp.int32),
            jax.ShapeDtypeStruct((T_TOT, 2), jnp.float32),
            jax.ShapeDtypeStruct((1, 1), jnp.float32),
        ],
        scratch_shapes=[
            pltpu.VMEM((T_BLK, N_EXP), jnp.float32),
            pltpu.VMEM((T_BLK, N_EXP), jnp.float32),
        ],
        compiler_params=pltpu.CompilerParams(
            dimension_semantics=("arbitrary",)),
    )(x_flat, gate_w)


_NW = NSC * NTILE
_CHUNK = NSLOT // _NW
_NROW = _CHUNK // 128


def _sc_take(a, idx):
    return lax.gather(
        a, idx[:, None],
        dimension_numbers=lax.GatherDimensionNumbers(
            offset_dims=(), collapsed_slice_dims=(0,), start_index_map=(0,)),
        slice_sizes=(1,),
        mode=lax.GatherScatterMode.PROMISE_IN_BOUNDS)


def _sc_splat(v, i):
    return _sc_take(v, jnp.full((16,), i, jnp.int32))


def _sc_treesum(v, lane):
    for d in (1, 2, 4, 8):
        v = v + _sc_take(v, lane ^ d)
    return v


def _sc_cumsum(v, lane):
    for d in (1, 2, 4, 8):
        g = _sc_take(v, jnp.maximum(lane - d, 0))
        v = v + jnp.where(lane >= d, g, 0)
    return v


def _sort_body(ef_hbm, wf_hbm, st_hbm, sw_hbm, posf_hbm, sex_hbm, ssk_hbm,
               ev, wv2, posb2, tokb2, cnt2, schedb, schedsk, sem1, sem2):
    wid = lax.axis_index("s") * NSC + lax.axis_index("c")
    lane = lax.iota(jnp.int32, 16)
    zeros_i = jnp.zeros((16,), jnp.int32)

    pltpu.sync_copy(ef_hbm, ev)
    sbase = wid * _CHUNK
    for r in range(_NROW):
        pltpu.sync_copy(wf_hbm.at[pl.ds(sbase + r * 128, 128)], wv2.at[r])

    for e in range(2 * N_EXP):
        cnt2[e] = zeros_i
    pgroups = zeros_i + wid * (_CHUNK // 16)

    def hbody(g, c):
        v = ev[pl.ds(g * 16, 16)]
        gv = zeros_i + g
        mpre = jnp.minimum(jnp.maximum(pgroups - gv, 0), 1)
        for e in range(N_EXP):
            inc = jnp.where(v == e, 1, 0)
            cnt2[e] = cnt2[e] + inc
            cnt2[N_EXP + e] = cnt2[N_EXP + e] + inc * mpre
        return c
    lax.fori_loop(0, NSLOT // 16, hbody, 0)

    c_tot = zeros_i
    pre = zeros_i
    for e in range(N_EXP):
        c_tot = c_tot + jnp.where(lane == e, _sc_treesum(cnt2[e], lane), 0)
        pre = pre + jnp.where(lane == e, _sc_treesum(cnt2[N_EXP + e], lane), 0)
    pad = (c_tot + (BT - 1)) & jnp.int32(-BT)
    incl = _sc_cumsum(pad, lane)
    poff = incl - pad
    base = poff + pre

    running = base
    for g in range(_CHUNK // 16):
        v = ev[pl.ds(sbase + g * 16, 16)]
        posv = zeros_i
        for e in range(N_EXP):
            mk = (v == e)
            ones = jnp.where(mk, 1, 0)
            cs = _sc_cumsum(ones, lane)
            be = _sc_splat(running, e)
            tot = _sc_splat(cs, 15)
            posv = jnp.where(mk, cs - 1 + be, posv)
            running = running + jnp.where(lane == e, tot, 0)
        posb2[g // 8, pl.ds((g % 8) * 16, 16)] = posv
        tokb2[g // 8, pl.ds((g % 8) * 16, 16)] = (lane + (sbase + g * 16)) >> 1

    for r in range(_NROW):
        pltpu.sync_copy(posb2.at[r], posf_hbm.at[pl.ds(sbase + r * 128, 128)])
        pltpu.async_copy(tokb2.at[r], st_hbm.at[posb2.at[r]], sem1).wait()
        pltpu.async_copy(wv2.at[r], sw_hbm.at[posb2.at[r]], sem2).wait()

    @pl.when(wid == 0)
    def _():
        end7 = _sc_splat(incl, N_EXP - 1)
        for vb in range(NSCHED // 16):
            bidx = lane + vb * 16
            v0 = bidx * BT
            exv = jnp.zeros((16,), jnp.int32)
            for e in range(N_EXP):
                end_e = _sc_splat(incl, e)
                exv = exv + jnp.where(v0 >= end_e, 1, 0)
            schedb[pl.ds(vb * 16, 16)] = jnp.minimum(exv, N_EXP - 1)
            schedsk[pl.ds(vb * 16, 16)] = jnp.where(v0 >= end7, 1, 0)
        pltpu.sync_copy(schedb, sex_hbm)
        pltpu.sync_copy(schedsk, ssk_hbm)


def _run_sort(ef, wf):
    mesh = plsc.VectorSubcoreMesh(core_axis_name="c", subcore_axis_name="s")
    return pl.kernel(
        _sort_body,
        out_type=[
            jax.ShapeDtypeStruct((P_PAD,), jnp.int32),
            jax.ShapeDtypeStruct((P_PAD,), jnp.float32),
            jax.ShapeDtypeStruct((NSLOT,), jnp.int32),
            jax.ShapeDtypeStruct((NSCHED,), jnp.int32),
            jax.ShapeDtypeStruct((NSCHED,), jnp.int32),
        ],
        mesh=mesh,
        scratch_types=[
            pltpu.VMEM((NSLOT,), jnp.int32),
            pltpu.VMEM((_NROW, 128), jnp.float32),
            pltpu.VMEM((_NROW, 128), jnp.int32),
            pltpu.VMEM((_NROW, 128), jnp.int32),
            pltpu.VMEM((2 * N_EXP, 16), jnp.int32),
            pltpu.VMEM((NSCHED,), jnp.int32),
            pltpu.VMEM((NSCHED,), jnp.int32),
            pltpu.SemaphoreType.DMA,
            pltpu.SemaphoreType.DMA,
        ],
    )(ef, wf)


_GROWS = P_PAD // (NSC * NTILE)


def _gather_body(st_hbm, x_hbm, xs_hbm, idx0, idx1, rows0, rows1,
                 sem0, sem1, wsem0, wsem1):
    wid = lax.axis_index("s") * NSC + lax.axis_index("c")
    base = wid * _GROWS
    nch = _GROWS // 32
    bufs = [(idx0, rows0, sem0, wsem0), (idx1, rows1, sem1, wsem1)]

    def fire(j):
        idxv, rowsv, sem, _ = bufs[j % 2]
        pltpu.sync_copy(st_hbm.at[pl.ds(base + j * 32, 32)], idxv)
        idxv[...] = jnp.minimum(jnp.maximum(idxv[...], 0), T_TOT - 1)
        return pltpu.async_copy(x_hbm.at[idxv], rowsv, sem)

    hs = {0: fire(0)}
    ws = {}
    for j in range(nch):
        if j + 1 < nch:
            if j - 1 >= 0:
                ws[j - 1].wait()
            hs[j + 1] = fire(j + 1)
        hs[j].wait()
        rowsv, wsem = bufs[j % 2][1], bufs[j % 2][3]
        ws[j] = pltpu.async_copy(rowsv, xs_hbm.at[pl.ds(base + j * 32, 32)],
                                 wsem)
    ws[nch - 2].wait()
    ws[nch - 1].wait()


def _run_gather(st, x_flat):
    mesh = plsc.VectorSubcoreMesh(core_axis_name="c", subcore_axis_name="s")
    return pl.kernel(
        _gather_body,
        out_type=jax.ShapeDtypeStruct((P_PAD, D_MODEL), jnp.float32),
        mesh=mesh,
        scratch_types=[
            pltpu.VMEM((32,), jnp.int32),
            pltpu.VMEM((32,), jnp.int32),
            pltpu.VMEM((32, D_MODEL), jnp.float32),
            pltpu.VMEM((32, D_MODEL), jnp.float32),
            pltpu.SemaphoreType.DMA,
            pltpu.SemaphoreType.DMA,
            pltpu.SemaphoreType.DMA,
            pltpu.SemaphoreType.DMA,
        ],
    )(st, x_flat)


def _ffn_body(sex_ref, ssk_ref, xs_ref, sw_ref, w1_ref, b1_ref, w2_ref,
              b2_ref, out_ref):
    b = pl.program_id(0)

    @pl.when(ssk_ref[b] == 0)
    def _():
        xb = xs_ref[...].astype(jnp.bfloat16)
        h = jnp.dot(xb, w1_ref[0], preferred_element_type=jnp.float32)
        h = h + b1_ref[0]
        h = h * jax.nn.sigmoid(h)
        o = jnp.dot(h.astype(jnp.bfloat16), w2_ref[0],
                    preferred_element_type=jnp.float32)
        out_ref[...] = (o + b2_ref[0]) * sw_ref[...]

    @pl.when(ssk_ref[b] != 0)
    def _():
        out_ref[...] = jnp.zeros_like(out_ref)


def _run_ffn(sex, ssk, xs, sw2d, w1_bf, b1r, w2_bf, b2r):
    grid_spec = pltpu.PrefetchScalarGridSpec(
        num_scalar_prefetch=2,
        grid=(NB,),
        in_specs=[
            pl.BlockSpec((BT, D_MODEL), lambda b, sex, ssk: (b, 0)),
            pl.BlockSpec((BT, 1), lambda b, sex, ssk: (b, 0)),
            pl.BlockSpec((1, D_MODEL, D_HIDDEN),
                         lambda b, sex, ssk: (sex[b], 0, 0)),
            pl.BlockSpec((1, 1, D_HIDDEN), lambda b, sex, ssk: (sex[b], 0, 0)),
            pl.BlockSpec((1, D_HIDDEN, D_MODEL),
                         lambda b, sex, ssk: (sex[b], 0, 0)),
            pl.BlockSpec((1, 1, D_MODEL), lambda b, sex, ssk: (sex[b], 0, 0)),
        ],
        out_specs=pl.BlockSpec((BT, D_MODEL), lambda b, sex, ssk: (b, 0)),
    )
    return pl.pallas_call(
        _ffn_body,
        grid_spec=grid_spec,
        out_shape=jax.ShapeDtypeStruct((P_PAD, D_MODEL), jnp.float32),
        compiler_params=pltpu.CompilerParams(
            dimension_semantics=("arbitrary",)),
    )(sex, ssk, xs, sw2d, w1_bf, b1r, w2_bf, b2r)


_CTOK = T_TOT // (NSC * NTILE)


def _combine_body(posf_hbm, os_hbm, out_hbm, pv0, pv1, rows0, rows1, outv,
                  sem0, sem1):
    wid = lax.axis_index("s") * NSC + lax.axis_index("c")
    tbase = wid * _CTOK
    pbase = wid * _CTOK * 2
    nch = _CTOK // 16
    bufs = [(pv0, rows0, sem0), (pv1, rows1, sem1)]

    def fire(j, buf):
        pv, rowsv, sem = buf
        pltpu.sync_copy(posf_hbm.at[pl.ds(pbase + j * 32, 32)], pv)
        return pltpu.async_copy(os_hbm.at[pv], rowsv, sem)

    hs = {0: fire(0, bufs[0])}
    for j in range(nch):
        if j + 1 < nch:
            hs[j + 1] = fire(j + 1, bufs[(j + 1) % 2])
        hs[j].wait()
        rowsv = bufs[j % 2][1]

        def row_body(r, carry):
            r2 = 2 * r
            for c in range(D_MODEL // 16):
                a = rowsv[r2, pl.ds(c * 16, 16)]
                bb = rowsv[r2 + 1, pl.ds(c * 16, 16)]
                outv[r, pl.ds(c * 16, 16)] = a + bb
            return carry

        lax.fori_loop(0, 16, row_body, 0)
        pltpu.sync_copy(outv, out_hbm.at[pl.ds(tbase + j * 16, 16)])


def _run_combine(posf, os):
    mesh = plsc.VectorSubcoreMesh(core_axis_name="c", subcore_axis_name="s")
    return pl.kernel(
        _combine_body,
        out_type=jax.ShapeDtypeStruct((T_TOT, D_MODEL), jnp.float32),
        mesh=mesh,
        scratch_types=[
            pltpu.VMEM((32,), jnp.int32),
            pltpu.VMEM((32,), jnp.int32),
            pltpu.VMEM((32, D_MODEL), jnp.float32),
            pltpu.VMEM((32, D_MODEL), jnp.float32),
            pltpu.VMEM((16, D_MODEL), jnp.float32),
            pltpu.SemaphoreType.DMA,
            pltpu.SemaphoreType.DMA,
        ],
    )(posf, os)


def kernel(x, gate_w, w1, b1, w2, b2):
    B, S, D = x.shape
    x_flat = x.reshape(B * S, D)

    eidx, wts, aux = _run_router(x_flat, gate_w)
    ef = eidx.reshape(NSLOT)
    wf = wts.reshape(NSLOT)

    st, sw, posf, sex, ssk = _run_sort(ef, wf)

    xs = _run_gather(st, x_flat)

    os_ = _run_ffn(sex, ssk, xs, sw.reshape(P_PAD, 1),
                   w1.astype(jnp.bfloat16), b1.reshape(N_EXP, 1, D_HIDDEN),
                   w2.astype(jnp.bfloat16), b2.reshape(N_EXP, 1, D_MODEL))

    out_flat = _run_combine(posf, os_)
    return out_flat.reshape(B, S, D), aux[0, 0]
